# baseline (speedup 1.0000x reference)
import functools

import jax
import jax.numpy as jnp
from jax import lax
from jax.experimental import pallas as pl
from jax.experimental.pallas import tpu as pltpu
from jax.experimental.pallas import tpu_sc as plsc

VOCAB = 1000000
EMBED = 64
BATCH = 16384

NUM_CORES = 2
NUM_SUBCORES = 16
LANES = 16
NW = NUM_CORES * NUM_SUBCORES
BPW = BATCH // NW
CH = 128
NCHUNK = BPW // CH

_mesh = plsc.VectorSubcoreMesh(core_axis_name="c", subcore_axis_name="s")


@functools.partial(
    pl.kernel,
    mesh=_mesh,
    out_type=jax.ShapeDtypeStruct((BATCH,), jnp.float32),
    scratch_types=[
        pltpu.VMEM((BPW,), jnp.int32),
        pltpu.VMEM((BPW,), jnp.int32),
        pltpu.VMEM((2, CH * EMBED), jnp.float32),
        pltpu.VMEM((2, CH * EMBED), jnp.float32),
        pltpu.VMEM((BPW + LANES,), jnp.float32),
        pltpu.SemaphoreType.DMA,
        pltpu.SemaphoreType.DMA,
    ],
    compiler_params=pltpu.CompilerParams(needs_layout_passes=False),
)
def _w2v_sc(center_hbm, context_hbm, in_emb_hbm, out_emb_hbm, out_hbm,
            cidx_v, xidx_v, arows_v, brows_v, out_v, sem0, sem1):
    wid = lax.axis_index("s") * NUM_CORES + lax.axis_index("c")
    base = wid * BPW

    pltpu.sync_copy(center_hbm.at[pl.ds(base, BPW)], cidx_v)
    pltpu.sync_copy(context_hbm.at[pl.ds(base, BPW)], xidx_v)

    sems = [sem0, sem1]
    last_lane = lax.iota(jnp.int32, LANES) == (LANES - 1)

    def issue_chunk(c, buf):
        sem = sems[buf]

        def issue_group(g, _):
            r0 = c * CH + g * LANES
            av = cidx_v[pl.ds(r0, LANES)]
            bv = xidx_v[pl.ds(r0, LANES)]
            for k in range(LANES):
                i = g * LANES + k
                pltpu.async_copy(in_emb_hbm.at[av[k]],
                                 arows_v.at[buf, pl.ds(i * EMBED, EMBED)], sem)
                pltpu.async_copy(out_emb_hbm.at[bv[k]],
                                 brows_v.at[buf, pl.ds(i * EMBED, EMBED)], sem)
            return 0

        lax.fori_loop(0, CH // LANES, issue_group, 0)

    def drain_chunk(buf):
        sem = sems[buf]

        def drain_row(i, _):
            pltpu.make_async_copy(in_emb_hbm.at[0],
                                  arows_v.at[buf, pl.ds(i * EMBED, EMBED)],
                                  sem).wait()
            pltpu.make_async_copy(out_emb_hbm.at[0],
                                  brows_v.at[buf, pl.ds(i * EMBED, EMBED)],
                                  sem).wait()
            return 0

        lax.fori_loop(0, CH, drain_row, 0, unroll=2)

    def compute_chunk(c, buf):
        def row_body(i, _):
            acc = jnp.zeros((LANES,), jnp.float32)
            for j in range(EMBED // LANES):
                a = arows_v[buf, pl.ds(i * EMBED + j * LANES, LANES)]
                b = brows_v[buf, pl.ds(i * EMBED + j * LANES, LANES)]
                acc = acc + a * b
            plsc.store_compressed(out_v.at[pl.ds(c * CH + i, LANES)],
                                  plsc.cumsum(acc), mask=last_lane)
            return 0

        lax.fori_loop(0, CH, row_body, 0, unroll=4)

    issue_chunk(0, 0)

    def loop_body(t, _):
        c0 = t * 2
        c1 = c0 + 1
        issue_chunk(c1, 1)
        drain_chunk(0)
        compute_chunk(c0, 0)

        @pl.when(c1 + 1 < NCHUNK)
        def _():
            issue_chunk(c1 + 1, 0)

        drain_chunk(1)
        compute_chunk(c1, 1)
        return 0

    lax.fori_loop(0, NCHUNK // 2, loop_body, 0)

    pltpu.sync_copy(out_v.at[pl.ds(0, BPW)], out_hbm.at[pl.ds(base, BPW)])


def kernel(center_idx, context_idx, in_embedding, out_embedding):
    return _w2v_sc(center_idx.astype(jnp.int32), context_idx.astype(jnp.int32),
                   in_embedding, out_embedding)


# speedup vs baseline: 1.0026x; 1.0026x over previous
import functools

import jax
import jax.numpy as jnp
from jax import lax
from jax.experimental import pallas as pl
from jax.experimental.pallas import tpu as pltpu
from jax.experimental.pallas import tpu_sc as plsc

VOCAB = 1000000
EMBED = 64
BATCH = 16384

NUM_CORES = 2
NUM_SUBCORES = 16
LANES = 16
NW = NUM_CORES * NUM_SUBCORES
BPW = BATCH // NW
CH = 128
NCHUNK = BPW // CH

_mesh = plsc.VectorSubcoreMesh(core_axis_name="c", subcore_axis_name="s")


@functools.partial(
    pl.kernel,
    mesh=_mesh,
    out_type=jax.ShapeDtypeStruct((BATCH,), jnp.float32),
    scratch_types=[
        pltpu.VMEM((BPW,), jnp.int32),
        pltpu.VMEM((BPW,), jnp.int32),
        pltpu.VMEM((2, CH * EMBED), jnp.float32),
        pltpu.VMEM((2, CH * EMBED), jnp.float32),
        pltpu.VMEM((BPW + LANES,), jnp.float32),
        pltpu.SemaphoreType.DMA,
        pltpu.SemaphoreType.DMA,
    ],
    compiler_params=pltpu.CompilerParams(needs_layout_passes=False),
)
def _w2v_sc(center_hbm, context_hbm, in_emb_hbm, out_emb_hbm, out_hbm,
            cidx_v, xidx_v, arows_v, brows_v, out_v, sem0, sem1):
    wid = lax.axis_index("s") * NUM_CORES + lax.axis_index("c")
    base = wid * BPW

    pltpu.sync_copy(center_hbm.at[pl.ds(base, BPW)], cidx_v)
    pltpu.sync_copy(context_hbm.at[pl.ds(base, BPW)], xidx_v)

    sems = [sem0, sem1]
    last_lane = lax.iota(jnp.int32, LANES) == (LANES - 1)

    def issue_chunk(c, buf):
        def issue_group(g, _):
            r0 = c * CH + g * LANES
            av = cidx_v[pl.ds(r0, LANES)]
            bv = xidx_v[pl.ds(r0, LANES)]
            for k in range(LANES):
                i = g * LANES + k
                pltpu.async_copy(in_emb_hbm.at[av[k]],
                                 arows_v.at[buf, pl.ds(i * EMBED, EMBED)],
                                 sems[0])
                pltpu.async_copy(out_emb_hbm.at[bv[k]],
                                 brows_v.at[buf, pl.ds(i * EMBED, EMBED)],
                                 sems[1])
            return 0

        lax.fori_loop(0, CH // LANES, issue_group, 0)

    def drain_chunk(buf):
        def drain_row(i, _):
            pltpu.make_async_copy(in_emb_hbm.at[0],
                                  arows_v.at[buf, pl.ds(i * EMBED, EMBED)],
                                  sems[0]).wait()
            pltpu.make_async_copy(out_emb_hbm.at[0],
                                  brows_v.at[buf, pl.ds(i * EMBED, EMBED)],
                                  sems[1]).wait()
            return 0

        lax.fori_loop(0, CH, drain_row, 0, unroll=2)

    def compute_chunk(c, buf):
        def row_body(i, _):
            acc = jnp.zeros((LANES,), jnp.float32)
            for j in range(EMBED // LANES):
                a = arows_v[buf, pl.ds(i * EMBED + j * LANES, LANES)]
                b = brows_v[buf, pl.ds(i * EMBED + j * LANES, LANES)]
                acc = acc + a * b
            plsc.store_compressed(out_v.at[pl.ds(c * CH + i, LANES)],
                                  plsc.cumsum(acc), mask=last_lane)
            return 0

        lax.fori_loop(0, CH, row_body, 0, unroll=4)

    def loop_body(c, _):
        issue_chunk(c, 0)
        drain_chunk(0)
        compute_chunk(c, 0)
        return 0

    lax.fori_loop(0, NCHUNK, loop_body, 0)

    pltpu.sync_copy(out_v.at[pl.ds(0, BPW)], out_hbm.at[pl.ds(base, BPW)])


def kernel(center_idx, context_idx, in_embedding, out_embedding):
    return _w2v_sc(center_idx.astype(jnp.int32), context_idx.astype(jnp.int32),
                   in_embedding, out_embedding)
